# baseline (speedup 1.0000x reference)
import jax
import jax.numpy as jnp
from jax.experimental import pallas as pl
from jax.experimental.pallas import tpu as pltpu

_B, _L, _D = 64, 256, 256
_H = 2 * _D
_C = 128
_EPS = 1e-5
_BB = 16
_SB = 4
_GRID = _B // _BB


def _hgt_kernel(feat_ref, labels_ref, wg1_ref, bg1_ref, wg2_ref, bg2_ref,
                w1_ref, b1_ref, gamma_ref, beta_ref, w2_ref, b2_ref,
                loss_ref, soft_ref, ro_ref):
    i = pl.program_id(0)

    fblk = feat_ref[...]
    wg1 = wg1_ref[...].astype(jnp.bfloat16)
    wg2t = wg2_ref[...].reshape(1, _H).astype(jnp.bfloat16)
    bg1 = bg1_ref[...].astype(jnp.bfloat16)
    sl = []
    for j in range(_BB // _SB):
        f2 = fblk[j * _SB:(j + 1) * _SB].reshape(_SB * _L, _D)
        z = jnp.dot(f2.astype(jnp.bfloat16), wg1,
                    preferred_element_type=jnp.float32).astype(jnp.bfloat16)
        g = jnp.maximum(z + bg1, 0)
        sl.append(jax.lax.dot_general(
            wg2t, g, dimension_numbers=(((1,), (1,)), ((), ())),
            preferred_element_type=jnp.float32))
    s = jnp.concatenate(sl, axis=1) if len(sl) > 1 else sl[0]
    s = (s + bg2_ref[...]).reshape(_BB, _L)
    s = s - jnp.max(s, axis=1, keepdims=True)
    e = jnp.exp(s)
    w = e / jnp.sum(e, axis=1, keepdims=True)
    ro = jax.lax.dot_general(
        w, fblk, dimension_numbers=(((1,), (1,)), ((0,), (0,))),
        preferred_element_type=jnp.float32)
    ro_ref[pl.ds(i * _BB, _BB), :] = ro

    @pl.when(i == _GRID - 1)
    def _decoder():
        x = jnp.dot(ro_ref[...], w1_ref[...],
                    preferred_element_type=jnp.float32) + b1_ref[...]
        x = jnp.maximum(x, 0.0)
        mean = jnp.mean(x, axis=0, keepdims=True)
        var = jnp.mean((x - mean) ** 2, axis=0, keepdims=True)
        xn = (x - mean) * jax.lax.rsqrt(var + _EPS) * gamma_ref[...] + beta_ref[...]
        logits = jnp.dot(xn, w2_ref[...],
                         preferred_element_type=jnp.float32) + b2_ref[...]
        m = jnp.max(logits, axis=1, keepdims=True)
        ex = jnp.exp(logits - m)
        se = jnp.sum(ex, axis=1, keepdims=True)
        soft_ref[...] = ex / se
        logp = logits - m - jnp.log(se)
        onehot = jax.lax.broadcasted_iota(jnp.int32, (_B, _C), 1) == labels_ref[...]
        lc = jnp.sum(jnp.where(onehot, logp, 0.0), axis=1, keepdims=True)
        loss_ref[...] = -jnp.mean(lc).reshape(1, 1)


def kernel(feat, labels, Wg1, bg1, Wg2, bg2, W1, b1, gamma, beta, W2, b2):
    labels2 = labels.astype(jnp.int32).reshape(_B, 1)
    loss, soft = pl.pallas_call(
        _hgt_kernel,
        grid=(_GRID,),
        in_specs=[
            pl.BlockSpec((_BB, _L, _D), lambda i: (i, 0, 0)),
            pl.BlockSpec((_B, 1), lambda i: (0, 0)),
            pl.BlockSpec((_D, _H), lambda i: (0, 0)),
            pl.BlockSpec((1, _H), lambda i: (0, 0)),
            pl.BlockSpec((_H, 1), lambda i: (0, 0)),
            pl.BlockSpec((1, 1), lambda i: (0, 0)),
            pl.BlockSpec((_D, _H), lambda i: (0, 0)),
            pl.BlockSpec((1, _H), lambda i: (0, 0)),
            pl.BlockSpec((1, _H), lambda i: (0, 0)),
            pl.BlockSpec((1, _H), lambda i: (0, 0)),
            pl.BlockSpec((_H, _C), lambda i: (0, 0)),
            pl.BlockSpec((1, _C), lambda i: (0, 0)),
        ],
        out_specs=[
            pl.BlockSpec((1, 1), lambda i: (0, 0)),
            pl.BlockSpec((_B, _C), lambda i: (0, 0)),
        ],
        out_shape=[
            jax.ShapeDtypeStruct((1, 1), jnp.float32),
            jax.ShapeDtypeStruct((_B, _C), jnp.float32),
        ],
        scratch_shapes=[pltpu.VMEM((_B, _D), jnp.float32)],
        compiler_params=pltpu.CompilerParams(
            dimension_semantics=("arbitrary",)),
    )(feat, labels2, Wg1, bg1.reshape(1, _H), Wg2, bg2.reshape(1, 1),
      W1, b1.reshape(1, _H), gamma.reshape(1, _H), beta.reshape(1, _H),
      W2, b2.reshape(1, _C))
    return loss[0, 0], soft


# speedup vs baseline: 1.0556x; 1.0556x over previous
import jax
import jax.numpy as jnp
from jax.experimental import pallas as pl
from jax.experimental.pallas import tpu as pltpu

_B, _L, _D = 64, 256, 256
_H = 2 * _D
_C = 128
_EPS = 1e-5
_BB = 16
_GRID = _B // _BB


def _hgt_kernel(feat_ref, labels_ref, wg1_ref, bg1_ref, wg2_ref, bg2_ref,
                w1_ref, b1_ref, gamma_ref, beta_ref, w2_ref, b2_ref,
                loss_ref, soft_ref, ro_ref):
    i = pl.program_id(0)

    fblk = feat_ref[...]
    f2 = fblk.reshape(_BB * _L, _D)
    g = jnp.dot(f2.astype(jnp.bfloat16), wg1_ref[...].astype(jnp.bfloat16),
                preferred_element_type=jnp.float32).astype(jnp.bfloat16)
    g = jnp.maximum(g + bg1_ref[...].astype(jnp.bfloat16), 0)
    s = jax.lax.dot_general(
        wg2_ref[...].reshape(1, _H).astype(jnp.bfloat16), g,
        dimension_numbers=(((1,), (1,)), ((), ())),
        preferred_element_type=jnp.float32)
    s = (s + bg2_ref[...]).reshape(_BB, _L)
    s = s - jnp.max(s, axis=1, keepdims=True)
    e = jnp.exp(s)
    w = e / jnp.sum(e, axis=1, keepdims=True)
    ro = jax.lax.dot_general(
        w, fblk, dimension_numbers=(((1,), (1,)), ((0,), (0,))),
        preferred_element_type=jnp.float32)
    ro_ref[pl.ds(i * _BB, _BB), :] = ro

    @pl.when(i == _GRID - 1)
    def _decoder():
        x = jnp.dot(ro_ref[...], w1_ref[...],
                    preferred_element_type=jnp.float32) + b1_ref[...]
        x = jnp.maximum(x, 0.0)
        mean = jnp.mean(x, axis=0, keepdims=True)
        var = jnp.mean((x - mean) ** 2, axis=0, keepdims=True)
        xn = (x - mean) * jax.lax.rsqrt(var + _EPS) * gamma_ref[...] + beta_ref[...]
        logits = jnp.dot(xn, w2_ref[...],
                         preferred_element_type=jnp.float32) + b2_ref[...]
        m = jnp.max(logits, axis=1, keepdims=True)
        ex = jnp.exp(logits - m)
        se = jnp.sum(ex, axis=1, keepdims=True)
        soft_ref[...] = ex / se
        logp = logits - m - jnp.log(se)
        onehot = jax.lax.broadcasted_iota(jnp.int32, (_B, _C), 1) == labels_ref[...]
        lc = jnp.sum(jnp.where(onehot, logp, 0.0), axis=1, keepdims=True)
        loss_ref[...] = -jnp.mean(lc).reshape(1, 1)


def kernel(feat, labels, Wg1, bg1, Wg2, bg2, W1, b1, gamma, beta, W2, b2):
    labels2 = labels.astype(jnp.int32).reshape(_B, 1)
    loss, soft = pl.pallas_call(
        _hgt_kernel,
        grid=(_GRID,),
        in_specs=[
            pl.BlockSpec((_BB, _L, _D), lambda i: (i, 0, 0)),
            pl.BlockSpec((_B, 1), lambda i: (0, 0)),
            pl.BlockSpec((_D, _H), lambda i: (0, 0)),
            pl.BlockSpec((1, _H), lambda i: (0, 0)),
            pl.BlockSpec((_H, 1), lambda i: (0, 0)),
            pl.BlockSpec((1, 1), lambda i: (0, 0)),
            pl.BlockSpec((_D, _H), lambda i: (0, 0)),
            pl.BlockSpec((1, _H), lambda i: (0, 0)),
            pl.BlockSpec((1, _H), lambda i: (0, 0)),
            pl.BlockSpec((1, _H), lambda i: (0, 0)),
            pl.BlockSpec((_H, _C), lambda i: (0, 0)),
            pl.BlockSpec((1, _C), lambda i: (0, 0)),
        ],
        out_specs=[
            pl.BlockSpec((1, 1), lambda i: (0, 0)),
            pl.BlockSpec((_B, _C), lambda i: (0, 0)),
        ],
        out_shape=[
            jax.ShapeDtypeStruct((1, 1), jnp.float32),
            jax.ShapeDtypeStruct((_B, _C), jnp.float32),
        ],
        scratch_shapes=[pltpu.VMEM((_B, _D), jnp.float32)],
        compiler_params=pltpu.CompilerParams(
            dimension_semantics=("arbitrary",)),
    )(feat, labels2, Wg1, bg1.reshape(1, _H), Wg2, bg2.reshape(1, 1),
      W1, b1.reshape(1, _H), gamma.reshape(1, _H), beta.reshape(1, _H),
      W2, b2.reshape(1, _C))
    return loss[0, 0], soft
